# Initial kernel scaffold; baseline (speedup 1.0000x reference)
import jax
import jax.numpy as jnp
from jax.experimental import pallas as pl


def kernel(x):
    raise NotImplementedError("write your pallas kernel here")


# speedup vs baseline: 19.0559x; 19.0559x over previous
import functools

import jax
import jax.numpy as jnp
from jax.experimental import pallas as pl

_K = 64


def _shift_right_cols(a, s):
    z = jnp.zeros((a.shape[0], s), dtype=a.dtype)
    return jnp.concatenate([z, a[:, : a.shape[1] - s]], axis=1)


def _shift_down_rows(a, s):
    z = jnp.zeros((s, a.shape[1]), dtype=a.dtype)
    return jnp.concatenate([z, a[: a.shape[0] - s, :]], axis=0)


def _topk_mask_kernel(x_ref, o_ref, *, kk):
    x = x_ref[...]
    v = jnp.maximum(x, 0.0)
    u = jnp.maximum(jax.lax.bitcast_convert_type(v, jnp.int32), 0)

    mx = jnp.max(u)

    def body(_, carry):
        lo, hi = carry
        mid = lo + (hi - lo) // 2
        c = jnp.sum((u >= mid).astype(jnp.int32))
        big = c >= kk
        return jnp.where(big, mid, lo), jnp.where(big, hi, mid)

    t_lo, _ = jax.lax.fori_loop(0, 31, body, (jnp.int32(0), mx + 1))
    T = t_lo

    gt = u > T
    eq = u == T
    n_gt = jnp.sum(gt.astype(jnp.int32))
    n_eq = jnp.sum(eq.astype(jnp.int32))

    def common(_):
        o_ref[...] = jnp.where(u >= T, v, 0.0)

    def rare(_):
        e = eq.astype(jnp.int32)
        c1 = e
        s = 1
        while s < e.shape[1]:
            c1 = c1 + _shift_right_cols(c1, s)
            s *= 2
        rowtot = c1[:, -1:]
        ro = rowtot
        s = 1
        while s < e.shape[0]:
            ro = ro + _shift_down_rows(ro, s)
            s *= 2
        rank = c1 + (ro - rowtot) - e
        keep = gt | (eq & (rank < (kk - n_gt)))
        o_ref[...] = jnp.where(keep, v, 0.0)

    jax.lax.cond(n_gt + n_eq == kk, common, rare, None)


def kernel(x):
    kk = _K * x.shape[0]
    return pl.pallas_call(
        functools.partial(_topk_mask_kernel, kk=kk),
        out_shape=jax.ShapeDtypeStruct(x.shape, x.dtype),
    )(x)
